# Initial kernel scaffold; baseline (speedup 1.0000x reference)
import jax
import jax.numpy as jnp
from jax.experimental import pallas as pl


def kernel(prediction, instances, labels, center_images):
    raise NotImplementedError("write your pallas kernel here")


# speedup vs baseline: 24.8338x; 24.8338x over previous
import functools

import numpy as np
import jax
import jax.numpy as jnp
from jax import lax
from jax.experimental import pallas as pl
from jax.experimental.pallas import tpu as pltpu
from jax.experimental.pallas import tpu_sc as plsc

_D, _H, _W = 16, 192, 192
_N = _D * _H * _W
_LANES = 128
_ROWS = _N // _LANES
_B = 2
_NID = 7
_PAIRS = _B * _NID
_K = 2048
_HIST = 2 * _PAIRS * _K
_NW = 32
_M = _PAIRS * _N
_MW = _M // _NW
_CH = 4096
_RCHUNK = 576
_NCH = _ROWS // _RCHUNK


def _coords(ch):
    rows = lax.broadcasted_iota(jnp.int32, (_RCHUNK, _LANES), 0)
    cols = lax.broadcasted_iota(jnp.int32, (_RCHUNK, _LANES), 1)
    n = (ch * _RCHUNK + rows) * _LANES + cols
    x = n % _W
    y = (n // _W) % _H
    z = n // (_W * _H)
    xm = x.astype(jnp.float32) * np.float32(1.0 / (256 - 1))
    ym = y.astype(jnp.float32) * np.float32(1.0 / (256 - 1))
    zm = z.astype(jnp.float32) * np.float32(1.0 / (32 - 1))
    return xm, ym, zm


def _stats_body(sig_ref, seed_ref, inst_ref, lab_ref, out_ref):
    xm, ym, zm = _coords(pl.program_id(1))
    s0 = sig_ref[0, 0]
    s1 = sig_ref[0, 1]
    s2 = sig_ref[0, 2]
    inst = inst_ref[0]
    vals = (xm, ym, zm, s0, s1, s2, s0 * s0, s1 * s1, s2 * s2)
    for iid in range(1, _NID + 1):
        mf = (inst == iid).astype(jnp.float32)
        r0 = (iid - 1) * 10
        out_ref[0, 0, r0, :] = jnp.sum(mf, axis=0)
        for j, v in enumerate(vals):
            out_ref[0, 0, r0 + 1 + j, :] = jnp.sum(v * mf, axis=0)
    seedm = jax.nn.sigmoid(seed_ref[0])
    bg = (lab_ref[0] == 0).astype(jnp.float32)
    out_ref[0, 0, 70, :] = jnp.sum(seedm * seedm * bg, axis=0)
    out_ref[0, 0, 71, :] = jnp.zeros((_LANES,), jnp.float32)


def _stats_call(sig3, seedlg, inst, lab):
    return pl.pallas_call(
        _stats_body,
        grid=(_B, _NCH),
        in_specs=[
            pl.BlockSpec((1, 3, _RCHUNK, _LANES), lambda b, c: (b, 0, c, 0)),
            pl.BlockSpec((1, _RCHUNK, _LANES), lambda b, c: (b, c, 0)),
            pl.BlockSpec((1, _RCHUNK, _LANES), lambda b, c: (b, c, 0)),
            pl.BlockSpec((1, _RCHUNK, _LANES), lambda b, c: (b, c, 0)),
        ],
        out_specs=pl.BlockSpec((1, 1, 72, _LANES), lambda b, c: (b, c, 0, 0)),
        out_shape=jax.ShapeDtypeStruct((_B, _NCH, 72, _LANES), jnp.float32),
    )(sig3, seedlg, inst, lab)


def _bins_body(emb_ref, seed_ref, inst_ref, prm_ref, bins_ref, sf_ref):
    xm, ym, zm = _coords(pl.program_id(1))
    e0 = jnp.tanh(emb_ref[0, 0]) + xm
    e1 = jnp.tanh(emb_ref[0, 1]) + ym
    e2 = jnp.tanh(emb_ref[0, 2]) + zm
    seedm = jax.nn.sigmoid(seed_ref[0])
    inst = inst_ref[0]
    pair0 = pl.program_id(0) * _NID
    for iid in range(1, _NID + 1):
        cx = prm_ref[0, iid - 1, 0]
        cy = prm_ref[0, iid - 1, 1]
        cz = prm_ref[0, iid - 1, 2]
        sx = prm_ref[0, iid - 1, 3]
        sy = prm_ref[0, iid - 1, 4]
        sz = prm_ref[0, iid - 1, 5]
        q = (e0 - cx) ** 2 * sx + (e1 - cy) ** 2 * sy + (e2 - cz) ** 2 * sz
        dd = jnp.exp(-q)
        mb = inst == iid
        e = jnp.where(mb, 2.0 - 2.0 * dd, 2.0 * dd)
        bin_ = jnp.minimum((e * np.float32(_K / 2.0)).astype(jnp.int32), _K - 1)
        flat = (jnp.where(mb, _PAIRS * _K, 0)
                + (pair0 + (iid - 1)) * _K + bin_)
        bins_ref[0, iid - 1] = flat
        sf_ref[0, 0, iid - 1, :] = jnp.sum(
            jnp.where(mb, (seedm - dd) ** 2, 0.0), axis=0)
    sf_ref[0, 0, _NID, :] = jnp.zeros((_LANES,), jnp.float32)


def _bins_call(emb3, seedlg, inst, prm):
    return pl.pallas_call(
        _bins_body,
        grid=(_B, _NCH),
        in_specs=[
            pl.BlockSpec((1, 3, _RCHUNK, _LANES), lambda b, c: (b, 0, c, 0)),
            pl.BlockSpec((1, _RCHUNK, _LANES), lambda b, c: (b, c, 0)),
            pl.BlockSpec((1, _RCHUNK, _LANES), lambda b, c: (b, c, 0)),
            pl.BlockSpec((1, _NID, 8), lambda b, c: (b, 0, 0),
                         memory_space=pltpu.SMEM),
        ],
        out_specs=[
            pl.BlockSpec((1, _NID, _RCHUNK, _LANES), lambda b, c: (b, 0, c, 0)),
            pl.BlockSpec((1, 1, _NID + 1, _LANES), lambda b, c: (b, c, 0, 0)),
        ],
        out_shape=[
            jax.ShapeDtypeStruct((_B, _NID, _ROWS, _LANES), jnp.int32),
            jax.ShapeDtypeStruct((_B, _NCH, _NID + 1, _LANES), jnp.float32),
        ],
    )(emb3, seedlg, inst, prm)


@functools.cache
def _make_sc_hist():
    mesh = plsc.VectorSubcoreMesh(core_axis_name="c", subcore_axis_name="s")

    @functools.partial(
        pl.kernel,
        out_type=jax.ShapeDtypeStruct((_NW, _HIST), jnp.int32),
        mesh=mesh,
        scratch_types=[
            pltpu.VMEM((_CH,), jnp.int32),
            pltpu.VMEM((_HIST,), jnp.int32),
        ],
        compiler_params=pltpu.CompilerParams(needs_layout_passes=False),
    )
    def _sc_hist_kernel(bins_hbm, out_hbm, stage, hist):
        wid = lax.axis_index("s") * 2 + lax.axis_index("c")
        base = wid * _MW
        zero16 = jnp.zeros((16,), jnp.int32)

        def zbody(i, carry):
            hist[pl.ds(i * 16, 16)] = zero16
            return carry

        lax.fori_loop(0, _HIST // 16, zbody, 0)

        def cbody(j, carry):
            pltpu.sync_copy(bins_hbm.at[pl.ds(base + j * _CH, _CH)], stage)

            def ibody(i, c2):
                idx = stage[pl.ds(i * 16, 16)]
                cnt, last = plsc.scan_count(idx)
                plsc.addupdate_scatter(hist, [idx], cnt, mask=last)
                return c2

            lax.fori_loop(0, _CH // 16, ibody, 0)
            return carry

        lax.fori_loop(0, _MW // _CH, cbody, 0)
        pltpu.sync_copy(hist, out_hbm.at[wid])

    return _sc_hist_kernel


def _sc_hist(bins_flat):
    return _make_sc_hist()(bins_flat)


def _lovasz_body(h_ref, out_ref, acc_ref):
    i = pl.program_id(0)

    @pl.when(i == 0)
    def _init():
        acc_ref[...] = jnp.zeros_like(acc_ref)

    acc_ref[...] += h_ref[0].astype(jnp.float32)

    @pl.when(i == _NW - 1)
    def _final():
        nn = acc_ref[0]
        pp = acc_ref[1]

        def suffix(x):
            s = 1
            while s < _K:
                x = x + jnp.concatenate(
                    [x[:, s:], jnp.zeros((_PAIRS, s), jnp.float32)], axis=1)
                s *= 2
            return x

        p_suf = suffix(pp)
        i_suf = suffix(pp + nn)
        ptot = p_suf[:, 0:1]
        jac = 1.0 - (ptot - p_suf) / jnp.maximum(ptot + i_suf - p_suf, 1.0)
        jend = 1.0 - ptot / jnp.maximum(ptot, 1.0)
        jnxt = jnp.concatenate([jac[:, 1:], jend], axis=1)
        kk = lax.broadcasted_iota(jnp.int32, (_PAIRS, _K), 1).astype(jnp.float32)
        ec = (2.0 * kk + 1.0) * np.float32(1.0 / _K)
        lv = jnp.sum(ec * (jac - jnxt), axis=1)
        out_ref[...] = jnp.broadcast_to(lv[:, None], (_PAIRS, _LANES))


def _lovasz_call(hists):
    return pl.pallas_call(
        _lovasz_body,
        grid=(_NW,),
        in_specs=[pl.BlockSpec((1, 2, _PAIRS, _K), lambda i: (i, 0, 0, 0))],
        out_specs=pl.BlockSpec((_PAIRS, _LANES), lambda i: (0, 0)),
        out_shape=jax.ShapeDtypeStruct((_PAIRS, _LANES), jnp.float32),
        scratch_shapes=[pltpu.VMEM((2, _PAIRS, _K), jnp.float32)],
    )(hists)


def kernel(prediction, instances, labels, center_images):
    sig3 = prediction[:, 3:6].reshape(_B, 3, _ROWS, _LANES)
    emb3 = prediction[:, 0:3].reshape(_B, 3, _ROWS, _LANES)
    seedlg = prediction[:, 6].reshape(_B, _ROWS, _LANES)
    inst = instances.reshape(_B, _ROWS, _LANES)
    lab = labels.reshape(_B, _ROWS, _LANES)

    stats = _stats_call(sig3, seedlg, inst, lab)
    sums = jnp.sum(stats, axis=(1, 3))
    per = sums[:, :70].reshape(_B, _NID, 10)
    bg_seed = sums[:, 70]
    cnt = per[..., 0]
    present = (cnt > 0).astype(jnp.float32)
    safe = jnp.maximum(cnt, 1.0)
    center = per[..., 1:4] / safe[..., None]
    s_mean = per[..., 4:7] / safe[..., None]
    sq = per[..., 7:10]
    var_pair = jnp.sum(sq - cnt[..., None] * s_mean ** 2, axis=-1) / (3.0 * safe)
    s_exp = jnp.exp(10.0 * s_mean)
    prm = jnp.concatenate(
        [center, s_exp, jnp.zeros((_B, _NID, 2), jnp.float32)], axis=-1)

    bins, sf = _bins_call(emb3, seedlg, inst, prm)
    seed_fg = jnp.sum(sf, axis=(1, 3))[:, :_NID]

    hists = _sc_hist(bins.reshape(_M))
    lov = _lovasz_call(hists.reshape(_NW, 2, _PAIRS, _K))[:, 0]
    lov = lov.reshape(_B, _NID)

    denom = jnp.maximum(jnp.sum(present, axis=1), 1.0)
    inst_b = jnp.sum(present * lov, axis=1) / denom
    var_b = jnp.sum(present * var_pair, axis=1) / denom
    seed_b = (bg_seed + jnp.sum(present * seed_fg, axis=1)) / np.float32(_N)
    li = jnp.sum(inst_b) * np.float32(1.0 / _B)
    lv = jnp.sum(var_b) * np.float32(10.0 / _B)
    ls = jnp.sum(seed_b) * np.float32(1.0 / _B)
    total = li + lv + ls
    return jnp.stack([li, lv, ls, total])


# speedup vs baseline: 28.5903x; 1.1513x over previous
import functools

import numpy as np
import jax
import jax.numpy as jnp
from jax import lax
from jax.experimental import pallas as pl
from jax.experimental.pallas import tpu as pltpu
from jax.experimental.pallas import tpu_sc as plsc

_D, _H, _W = 16, 192, 192
_N = _D * _H * _W
_LANES = 128
_ROWS = _N // _LANES
_B = 2
_NID = 7
_PAIRS = _B * _NID
_K = 2048
_HIST = 2 * _PAIRS * _K
_NW = 32
_M = _PAIRS * _N
_MW = _M // _NW
_CH = 4032
_NCHK = 64
_UNR = 6
_RCHUNK = 576
_NCH = _ROWS // _RCHUNK


def _coords(ch):
    rows = lax.broadcasted_iota(jnp.int32, (_RCHUNK, _LANES), 0)
    cols = lax.broadcasted_iota(jnp.int32, (_RCHUNK, _LANES), 1)
    n = (ch * _RCHUNK + rows) * _LANES + cols
    x = n % _W
    y = (n // _W) % _H
    z = n // (_W * _H)
    xm = x.astype(jnp.float32) * np.float32(1.0 / (256 - 1))
    ym = y.astype(jnp.float32) * np.float32(1.0 / (256 - 1))
    zm = z.astype(jnp.float32) * np.float32(1.0 / (32 - 1))
    return xm, ym, zm


def _stats_body(sig_ref, seed_ref, inst_ref, lab_ref, out_ref):
    xm, ym, zm = _coords(pl.program_id(1))
    s0 = sig_ref[0, 0]
    s1 = sig_ref[0, 1]
    s2 = sig_ref[0, 2]
    inst = inst_ref[0]
    vals = (xm, ym, zm, s0, s1, s2, s0 * s0, s1 * s1, s2 * s2)
    for iid in range(1, _NID + 1):
        mf = (inst == iid).astype(jnp.float32)
        r0 = (iid - 1) * 10
        out_ref[0, 0, r0, :] = jnp.sum(mf, axis=0)
        for j, v in enumerate(vals):
            out_ref[0, 0, r0 + 1 + j, :] = jnp.sum(v * mf, axis=0)
    seedm = jax.nn.sigmoid(seed_ref[0])
    bg = (lab_ref[0] == 0).astype(jnp.float32)
    out_ref[0, 0, 70, :] = jnp.sum(seedm * seedm * bg, axis=0)
    out_ref[0, 0, 71, :] = jnp.zeros((_LANES,), jnp.float32)


def _stats_call(sig3, seedlg, inst, lab):
    return pl.pallas_call(
        _stats_body,
        grid=(_B, _NCH),
        in_specs=[
            pl.BlockSpec((1, 3, _RCHUNK, _LANES), lambda b, c: (b, 0, c, 0)),
            pl.BlockSpec((1, _RCHUNK, _LANES), lambda b, c: (b, c, 0)),
            pl.BlockSpec((1, _RCHUNK, _LANES), lambda b, c: (b, c, 0)),
            pl.BlockSpec((1, _RCHUNK, _LANES), lambda b, c: (b, c, 0)),
        ],
        out_specs=pl.BlockSpec((1, 1, 72, _LANES), lambda b, c: (b, c, 0, 0)),
        out_shape=jax.ShapeDtypeStruct((_B, _NCH, 72, _LANES), jnp.float32),
    )(sig3, seedlg, inst, lab)


def _bins_body(emb_ref, seed_ref, inst_ref, prm_ref, bins_ref, sf_ref):
    xm, ym, zm = _coords(pl.program_id(1))
    e0 = jnp.tanh(emb_ref[0, 0]) + xm
    e1 = jnp.tanh(emb_ref[0, 1]) + ym
    e2 = jnp.tanh(emb_ref[0, 2]) + zm
    seedm = jax.nn.sigmoid(seed_ref[0])
    inst = inst_ref[0]
    pair0 = pl.program_id(0) * _NID
    for iid in range(1, _NID + 1):
        cx = prm_ref[0, iid - 1, 0]
        cy = prm_ref[0, iid - 1, 1]
        cz = prm_ref[0, iid - 1, 2]
        sx = prm_ref[0, iid - 1, 3]
        sy = prm_ref[0, iid - 1, 4]
        sz = prm_ref[0, iid - 1, 5]
        q = (e0 - cx) ** 2 * sx + (e1 - cy) ** 2 * sy + (e2 - cz) ** 2 * sz
        dd = jnp.exp(-q)
        mb = inst == iid
        e = jnp.where(mb, 2.0 - 2.0 * dd, 2.0 * dd)
        bin_ = jnp.minimum((e * np.float32(_K / 2.0)).astype(jnp.int32), _K - 1)
        flat = (jnp.where(mb, _PAIRS * _K, 0)
                + (pair0 + (iid - 1)) * _K + bin_)
        bins_ref[0, iid - 1] = flat
        sf_ref[0, 0, iid - 1, :] = jnp.sum(
            jnp.where(mb, (seedm - dd) ** 2, 0.0), axis=0)
    sf_ref[0, 0, _NID, :] = jnp.zeros((_LANES,), jnp.float32)


def _bins_call(emb3, seedlg, inst, prm):
    return pl.pallas_call(
        _bins_body,
        grid=(_B, _NCH),
        in_specs=[
            pl.BlockSpec((1, 3, _RCHUNK, _LANES), lambda b, c: (b, 0, c, 0)),
            pl.BlockSpec((1, _RCHUNK, _LANES), lambda b, c: (b, c, 0)),
            pl.BlockSpec((1, _RCHUNK, _LANES), lambda b, c: (b, c, 0)),
            pl.BlockSpec((1, _NID, 8), lambda b, c: (b, 0, 0),
                         memory_space=pltpu.SMEM),
        ],
        out_specs=[
            pl.BlockSpec((1, _NID, _RCHUNK, _LANES), lambda b, c: (b, 0, c, 0)),
            pl.BlockSpec((1, 1, _NID + 1, _LANES), lambda b, c: (b, c, 0, 0)),
        ],
        out_shape=[
            jax.ShapeDtypeStruct((_B, _NID, _ROWS, _LANES), jnp.int32),
            jax.ShapeDtypeStruct((_B, _NCH, _NID + 1, _LANES), jnp.float32),
        ],
    )(emb3, seedlg, inst, prm)


@functools.cache
def _make_sc_hist():
    mesh = plsc.VectorSubcoreMesh(core_axis_name="c", subcore_axis_name="s")

    @functools.partial(
        pl.kernel,
        out_type=jax.ShapeDtypeStruct((_NW, _HIST), jnp.int32),
        mesh=mesh,
        scratch_types=[
            pltpu.VMEM((_CH,), jnp.int32),
            pltpu.VMEM((_CH,), jnp.int32),
            pltpu.VMEM((_HIST,), jnp.int32),
            pltpu.SemaphoreType.DMA,
            pltpu.SemaphoreType.DMA,
        ],
        compiler_params=pltpu.CompilerParams(needs_layout_passes=False),
    )
    def _sc_hist_kernel(bins_hbm, out_hbm, stage0, stage1, hist, sem0, sem1):
        wid = lax.axis_index("s") * 2 + lax.axis_index("c")
        base = wid * _MW
        stages = (stage0, stage1)
        sems = (sem0, sem1)
        zero16 = jnp.zeros((16,), jnp.int32)

        def zbody(i, carry):
            for u in range(8):
                hist[pl.ds((i * 8 + u) * 16, 16)] = zero16
            return carry

        lax.fori_loop(0, _HIST // 16 // 8, zbody, 0)

        def start(j, b):
            pltpu.async_copy(
                bins_hbm.at[pl.ds(base + j * _CH, _CH)], stages[b], sems[b])

        def wait(b):
            pltpu.make_async_copy(
                bins_hbm.at[pl.ds(base, _CH)], stages[b], sems[b]).wait()

        start(0, 0)

        def obody(jo, carry):
            for b in range(2):
                j = jo * 2 + b
                wait(b)
                nxt = j + 1

                @pl.when(nxt < _NCHK)
                def _pref():
                    start(nxt, 1 - b)

                def ibody(i, c2):
                    for u in range(_UNR):
                        off = (i * _UNR + u) * 16
                        idx = stages[b][pl.ds(off, 16)]
                        cnt, last = plsc.scan_count(idx)
                        plsc.addupdate_scatter(hist, [idx], cnt, mask=last)
                    return c2

                lax.fori_loop(0, _CH // 16 // _UNR, ibody, 0)
            return carry

        lax.fori_loop(0, _NCHK // 2, obody, 0)
        pltpu.sync_copy(hist, out_hbm.at[wid])

    return _sc_hist_kernel


def _sc_hist(bins_flat):
    return _make_sc_hist()(bins_flat)


def _lovasz_body(h_ref, out_ref, acc_ref):
    i = pl.program_id(0)

    @pl.when(i == 0)
    def _init():
        acc_ref[...] = jnp.zeros_like(acc_ref)

    acc_ref[...] += h_ref[0].astype(jnp.float32)

    @pl.when(i == _NW - 1)
    def _final():
        nn = acc_ref[0]
        pp = acc_ref[1]

        def suffix(x):
            s = 1
            while s < _K:
                x = x + jnp.concatenate(
                    [x[:, s:], jnp.zeros((_PAIRS, s), jnp.float32)], axis=1)
                s *= 2
            return x

        p_suf = suffix(pp)
        i_suf = suffix(pp + nn)
        ptot = p_suf[:, 0:1]
        jac = 1.0 - (ptot - p_suf) / jnp.maximum(ptot + i_suf - p_suf, 1.0)
        jend = 1.0 - ptot / jnp.maximum(ptot, 1.0)
        jnxt = jnp.concatenate([jac[:, 1:], jend], axis=1)
        kk = lax.broadcasted_iota(jnp.int32, (_PAIRS, _K), 1).astype(jnp.float32)
        ec = (2.0 * kk + 1.0) * np.float32(1.0 / _K)
        lv = jnp.sum(ec * (jac - jnxt), axis=1)
        out_ref[...] = jnp.broadcast_to(lv[:, None], (_PAIRS, _LANES))


def _lovasz_call(hists):
    return pl.pallas_call(
        _lovasz_body,
        grid=(_NW,),
        in_specs=[pl.BlockSpec((1, 2, _PAIRS, _K), lambda i: (i, 0, 0, 0))],
        out_specs=pl.BlockSpec((_PAIRS, _LANES), lambda i: (0, 0)),
        out_shape=jax.ShapeDtypeStruct((_PAIRS, _LANES), jnp.float32),
        scratch_shapes=[pltpu.VMEM((2, _PAIRS, _K), jnp.float32)],
    )(hists)


def kernel(prediction, instances, labels, center_images):
    sig3 = prediction[:, 3:6].reshape(_B, 3, _ROWS, _LANES)
    emb3 = prediction[:, 0:3].reshape(_B, 3, _ROWS, _LANES)
    seedlg = prediction[:, 6].reshape(_B, _ROWS, _LANES)
    inst = instances.reshape(_B, _ROWS, _LANES)
    lab = labels.reshape(_B, _ROWS, _LANES)

    stats = _stats_call(sig3, seedlg, inst, lab)
    sums = jnp.sum(stats, axis=(1, 3))
    per = sums[:, :70].reshape(_B, _NID, 10)
    bg_seed = sums[:, 70]
    cnt = per[..., 0]
    present = (cnt > 0).astype(jnp.float32)
    safe = jnp.maximum(cnt, 1.0)
    center = per[..., 1:4] / safe[..., None]
    s_mean = per[..., 4:7] / safe[..., None]
    sq = per[..., 7:10]
    var_pair = jnp.sum(sq - cnt[..., None] * s_mean ** 2, axis=-1) / (3.0 * safe)
    s_exp = jnp.exp(10.0 * s_mean)
    prm = jnp.concatenate(
        [center, s_exp, jnp.zeros((_B, _NID, 2), jnp.float32)], axis=-1)

    bins, sf = _bins_call(emb3, seedlg, inst, prm)
    seed_fg = jnp.sum(sf, axis=(1, 3))[:, :_NID]

    hists = _sc_hist(bins.reshape(_M))
    lov = _lovasz_call(hists.reshape(_NW, 2, _PAIRS, _K))[:, 0]
    lov = lov.reshape(_B, _NID)

    denom = jnp.maximum(jnp.sum(present, axis=1), 1.0)
    inst_b = jnp.sum(present * lov, axis=1) / denom
    var_b = jnp.sum(present * var_pair, axis=1) / denom
    seed_b = (bg_seed + jnp.sum(present * seed_fg, axis=1)) / np.float32(_N)
    li = jnp.sum(inst_b) * np.float32(1.0 / _B)
    lv = jnp.sum(var_b) * np.float32(10.0 / _B)
    ls = jnp.sum(seed_b) * np.float32(1.0 / _B)
    total = li + lv + ls
    return jnp.stack([li, lv, ls, total])


# speedup vs baseline: 47.8242x; 1.6727x over previous
import functools

import numpy as np
import jax
import jax.numpy as jnp
from jax import lax
from jax.experimental import pallas as pl
from jax.experimental.pallas import tpu as pltpu
from jax.experimental.pallas import tpu_sc as plsc

_D, _H, _W = 16, 192, 192
_N = _D * _H * _W
_LANES = 128
_ROWS = _N // _LANES
_B = 2
_NID = 7
_PAIRS = _B * _NID
_K = 2048
_HIST = 2 * _PAIRS * _K
_NW = 32
_M = _PAIRS * _N
_MW = _M // _NW
_CH = 4032
_NCHK = 64
_UNR = 6
_RCHUNK = 576
_NCH = _ROWS // _RCHUNK


def _coords(ch):
    rows = lax.broadcasted_iota(jnp.int32, (_RCHUNK, _LANES), 0)
    cols = lax.broadcasted_iota(jnp.int32, (_RCHUNK, _LANES), 1)
    n = (ch * _RCHUNK + rows) * _LANES + cols
    x = n % _W
    y = (n // _W) % _H
    z = n // (_W * _H)
    xm = x.astype(jnp.float32) * np.float32(1.0 / (256 - 1))
    ym = y.astype(jnp.float32) * np.float32(1.0 / (256 - 1))
    zm = z.astype(jnp.float32) * np.float32(1.0 / (32 - 1))
    return xm, ym, zm


def _stats_body(sig_ref, seed_ref, inst_ref, lab_ref, out_ref):
    xm, ym, zm = _coords(pl.program_id(1))
    s0 = sig_ref[0, 0]
    s1 = sig_ref[0, 1]
    s2 = sig_ref[0, 2]
    inst = inst_ref[0]
    vals = (xm, ym, zm, s0, s1, s2, s0 * s0, s1 * s1, s2 * s2)
    for iid in range(1, _NID + 1):
        mf = (inst == iid).astype(jnp.float32)
        r0 = (iid - 1) * 10
        out_ref[0, 0, r0, :] = jnp.sum(mf, axis=0)
        for j, v in enumerate(vals):
            out_ref[0, 0, r0 + 1 + j, :] = jnp.sum(v * mf, axis=0)
    seedm = jax.nn.sigmoid(seed_ref[0])
    bg = (lab_ref[0] == 0).astype(jnp.float32)
    out_ref[0, 0, 70, :] = jnp.sum(seedm * seedm * bg, axis=0)
    out_ref[0, 0, 71, :] = jnp.zeros((_LANES,), jnp.float32)


def _stats_call(sig3, seedlg, inst, lab):
    return pl.pallas_call(
        _stats_body,
        grid=(_B, _NCH),
        in_specs=[
            pl.BlockSpec((1, 3, _RCHUNK, _LANES), lambda b, c: (b, 0, c, 0)),
            pl.BlockSpec((1, _RCHUNK, _LANES), lambda b, c: (b, c, 0)),
            pl.BlockSpec((1, _RCHUNK, _LANES), lambda b, c: (b, c, 0)),
            pl.BlockSpec((1, _RCHUNK, _LANES), lambda b, c: (b, c, 0)),
        ],
        out_specs=pl.BlockSpec((1, 1, 72, _LANES), lambda b, c: (b, c, 0, 0)),
        out_shape=jax.ShapeDtypeStruct((_B, _NCH, 72, _LANES), jnp.float32),
    )(sig3, seedlg, inst, lab)


def _bins_body(emb_ref, seed_ref, inst_ref, prm_ref, bins_ref, sf_ref):
    xm, ym, zm = _coords(pl.program_id(1))
    e0 = jnp.tanh(emb_ref[0, 0]) + xm
    e1 = jnp.tanh(emb_ref[0, 1]) + ym
    e2 = jnp.tanh(emb_ref[0, 2]) + zm
    seedm = jax.nn.sigmoid(seed_ref[0])
    inst = inst_ref[0]
    pair0 = pl.program_id(0) * _NID
    for iid in range(1, _NID + 1):
        cx = prm_ref[0, iid - 1, 0]
        cy = prm_ref[0, iid - 1, 1]
        cz = prm_ref[0, iid - 1, 2]
        sx = prm_ref[0, iid - 1, 3]
        sy = prm_ref[0, iid - 1, 4]
        sz = prm_ref[0, iid - 1, 5]
        q = (e0 - cx) ** 2 * sx + (e1 - cy) ** 2 * sy + (e2 - cz) ** 2 * sz
        dd = jnp.exp(-q)
        mb = inst == iid
        e = jnp.where(mb, 2.0 - 2.0 * dd, 2.0 * dd)
        bin_ = jnp.minimum((e * np.float32(_K / 2.0)).astype(jnp.int32), _K - 1)
        flat = (jnp.where(mb, _PAIRS * _K, 0)
                + (pair0 + (iid - 1)) * _K + bin_)
        bins_ref[0, iid - 1] = flat
        sf_ref[0, 0, iid - 1, :] = jnp.sum(
            jnp.where(mb, (seedm - dd) ** 2, 0.0), axis=0)
    sf_ref[0, 0, _NID, :] = jnp.zeros((_LANES,), jnp.float32)


def _bins_call(emb3, seedlg, inst, prm):
    return pl.pallas_call(
        _bins_body,
        grid=(_B, _NCH),
        in_specs=[
            pl.BlockSpec((1, 3, _RCHUNK, _LANES), lambda b, c: (b, 0, c, 0)),
            pl.BlockSpec((1, _RCHUNK, _LANES), lambda b, c: (b, c, 0)),
            pl.BlockSpec((1, _RCHUNK, _LANES), lambda b, c: (b, c, 0)),
            pl.BlockSpec((1, _NID, 8), lambda b, c: (b, 0, 0),
                         memory_space=pltpu.SMEM),
        ],
        out_specs=[
            pl.BlockSpec((1, _NID, _RCHUNK, _LANES), lambda b, c: (b, 0, c, 0)),
            pl.BlockSpec((1, 1, _NID + 1, _LANES), lambda b, c: (b, c, 0, 0)),
        ],
        out_shape=[
            jax.ShapeDtypeStruct((_B, _NID, _ROWS, _LANES), jnp.int32),
            jax.ShapeDtypeStruct((_B, _NCH, _NID + 1, _LANES), jnp.float32),
        ],
    )(emb3, seedlg, inst, prm)


@functools.cache
def _make_sc_hist():
    mesh = plsc.VectorSubcoreMesh(core_axis_name="c", subcore_axis_name="s")

    @functools.partial(
        pl.kernel,
        out_type=jax.ShapeDtypeStruct((_NW, _HIST), jnp.int32),
        mesh=mesh,
        scratch_types=[
            pltpu.VMEM((_CH,), jnp.int32),
            pltpu.VMEM((_CH,), jnp.int32),
            pltpu.VMEM((_HIST,), jnp.int32),
            pltpu.SemaphoreType.DMA,
            pltpu.SemaphoreType.DMA,
        ],
        compiler_params=pltpu.CompilerParams(needs_layout_passes=False),
    )
    def _sc_hist_kernel(bins_hbm, out_hbm, stage0, stage1, hist, sem0, sem1):
        wid = lax.axis_index("s") * 2 + lax.axis_index("c")
        base = wid * _MW
        stages = (stage0, stage1)
        sems = (sem0, sem1)
        zero16 = jnp.zeros((16,), jnp.int32)

        def zbody(i, carry):
            for u in range(8):
                hist[pl.ds((i * 8 + u) * 16, 16)] = zero16
            return carry

        lax.fori_loop(0, _HIST // 16 // 8, zbody, 0)

        def start(j, b):
            pltpu.async_copy(
                bins_hbm.at[pl.ds(base + j * _CH, _CH)], stages[b], sems[b])

        def wait(b):
            pltpu.make_async_copy(
                bins_hbm.at[pl.ds(base, _CH)], stages[b], sems[b]).wait()

        start(0, 0)

        def obody(jo, carry):
            for b in range(2):
                j = jo * 2 + b
                wait(b)
                nxt = j + 1

                @pl.when(nxt < _NCHK)
                def _pref():
                    start(nxt, 1 - b)

                @plsc.parallel_loop(0, _CH // 16, unroll=_UNR)
                def _scatter(i):
                    idx = stages[b][pl.ds(i * 16, 16)]
                    cnt, last = plsc.scan_count(idx)
                    plsc.addupdate_scatter(hist, [idx], cnt, mask=last)
            return carry

        lax.fori_loop(0, _NCHK // 2, obody, 0)
        pltpu.sync_copy(hist, out_hbm.at[wid])

    return _sc_hist_kernel


def _sc_hist(bins_flat):
    return _make_sc_hist()(bins_flat)


def _lovasz_body(h_ref, out_ref, acc_ref):
    i = pl.program_id(0)

    @pl.when(i == 0)
    def _init():
        acc_ref[...] = jnp.zeros_like(acc_ref)

    acc_ref[...] += h_ref[0].astype(jnp.float32)

    @pl.when(i == _NW - 1)
    def _final():
        nn = acc_ref[0]
        pp = acc_ref[1]

        def suffix(x):
            s = 1
            while s < _K:
                x = x + jnp.concatenate(
                    [x[:, s:], jnp.zeros((_PAIRS, s), jnp.float32)], axis=1)
                s *= 2
            return x

        p_suf = suffix(pp)
        i_suf = suffix(pp + nn)
        ptot = p_suf[:, 0:1]
        jac = 1.0 - (ptot - p_suf) / jnp.maximum(ptot + i_suf - p_suf, 1.0)
        jend = 1.0 - ptot / jnp.maximum(ptot, 1.0)
        jnxt = jnp.concatenate([jac[:, 1:], jend], axis=1)
        kk = lax.broadcasted_iota(jnp.int32, (_PAIRS, _K), 1).astype(jnp.float32)
        ec = (2.0 * kk + 1.0) * np.float32(1.0 / _K)
        lv = jnp.sum(ec * (jac - jnxt), axis=1)
        out_ref[...] = jnp.broadcast_to(lv[:, None], (_PAIRS, _LANES))


def _lovasz_call(hists):
    return pl.pallas_call(
        _lovasz_body,
        grid=(_NW,),
        in_specs=[pl.BlockSpec((1, 2, _PAIRS, _K), lambda i: (i, 0, 0, 0))],
        out_specs=pl.BlockSpec((_PAIRS, _LANES), lambda i: (0, 0)),
        out_shape=jax.ShapeDtypeStruct((_PAIRS, _LANES), jnp.float32),
        scratch_shapes=[pltpu.VMEM((2, _PAIRS, _K), jnp.float32)],
    )(hists)


def kernel(prediction, instances, labels, center_images):
    sig3 = prediction[:, 3:6].reshape(_B, 3, _ROWS, _LANES)
    emb3 = prediction[:, 0:3].reshape(_B, 3, _ROWS, _LANES)
    seedlg = prediction[:, 6].reshape(_B, _ROWS, _LANES)
    inst = instances.reshape(_B, _ROWS, _LANES)
    lab = labels.reshape(_B, _ROWS, _LANES)

    stats = _stats_call(sig3, seedlg, inst, lab)
    sums = jnp.sum(stats, axis=(1, 3))
    per = sums[:, :70].reshape(_B, _NID, 10)
    bg_seed = sums[:, 70]
    cnt = per[..., 0]
    present = (cnt > 0).astype(jnp.float32)
    safe = jnp.maximum(cnt, 1.0)
    center = per[..., 1:4] / safe[..., None]
    s_mean = per[..., 4:7] / safe[..., None]
    sq = per[..., 7:10]
    var_pair = jnp.sum(sq - cnt[..., None] * s_mean ** 2, axis=-1) / (3.0 * safe)
    s_exp = jnp.exp(10.0 * s_mean)
    prm = jnp.concatenate(
        [center, s_exp, jnp.zeros((_B, _NID, 2), jnp.float32)], axis=-1)

    bins, sf = _bins_call(emb3, seedlg, inst, prm)
    seed_fg = jnp.sum(sf, axis=(1, 3))[:, :_NID]

    hists = _sc_hist(bins.reshape(_M))
    lov = _lovasz_call(hists.reshape(_NW, 2, _PAIRS, _K))[:, 0]
    lov = lov.reshape(_B, _NID)

    denom = jnp.maximum(jnp.sum(present, axis=1), 1.0)
    inst_b = jnp.sum(present * lov, axis=1) / denom
    var_b = jnp.sum(present * var_pair, axis=1) / denom
    seed_b = (bg_seed + jnp.sum(present * seed_fg, axis=1)) / np.float32(_N)
    li = jnp.sum(inst_b) * np.float32(1.0 / _B)
    lv = jnp.sum(var_b) * np.float32(10.0 / _B)
    ls = jnp.sum(seed_b) * np.float32(1.0 / _B)
    total = li + lv + ls
    return jnp.stack([li, lv, ls, total])
